# baseline (speedup 1.0000x reference)
import functools

import jax
import jax.numpy as jnp
from jax import lax
from jax.experimental import pallas as pl
from jax.experimental.pallas import tpu as pltpu
from jax.experimental.pallas import tpu_sc as plsc

_NUM_EMB = 8192
_DIM = 256
_COMMIT = 0.25

_MBLK = 512
_NBLK = 512
_NSTEPS = _NUM_EMB // _NBLK


def _argmin_body(x_ref, xsq_ref, emb_ref, ysq_ref, codes_ref, dsum_ref,
                 acc_ref, rm_s, rj_s):
    i = pl.program_id(0)
    last = pl.num_programs(0) - 1

    @pl.when(i > 0)
    def _():
        b = lax.rem(i - 1, 2)
        rm = rm_s[b]
        rj = rj_s[b]
        run_min = jnp.min(rm, axis=1, keepdims=True)
        lanes = lax.broadcasted_iota(jnp.int32, rm.shape, 1)
        code_vec = rj * _NBLK + lanes
        run_idx = jnp.min(jnp.where(rm == run_min, code_vec, _NUM_EMB),
                          axis=1, keepdims=True)
        codes_ref[...] = run_idx
        prev = jnp.where(i == 1, 0.0, acc_ref[0, 0])
        acc_ref[0, 0] = prev + jnp.sum(run_min)

        @pl.when(i == last)
        def _():
            dsum_ref[0, 0] = acc_ref[0, 0]

    @pl.when(i < last)
    def _():
        x2 = x_ref[...]
        xsq = xsq_ref[...]

        def dist(j):
            e = emb_ref[pl.ds(j * _NBLK, _NBLK), :]
            ysq = ysq_ref[:, pl.ds(j * _NBLK, _NBLK)]
            s2 = lax.dot_general(x2, e, (((1,), (1,)), ((), ())),
                                 preferred_element_type=jnp.float32)
            return (xsq + ysq) - s2

        rm = None
        rj = None
        for k in range(_NSTEPS // 4):
            d0 = dist(4 * k)
            d1 = dist(4 * k + 1)
            d2 = dist(4 * k + 2)
            d3 = dist(4 * k + 3)
            m01 = jnp.minimum(d0, d1)
            j01 = jnp.where(d1 < d0, 4 * k + 1, 4 * k).astype(jnp.int32)
            m23 = jnp.minimum(d2, d3)
            j23 = jnp.where(d3 < d2, 4 * k + 3, 4 * k + 2).astype(jnp.int32)
            m4 = jnp.minimum(m01, m23)
            j4 = jnp.where(m23 < m01, j23, j01)
            if rm is None:
                rm, rj = m4, j4
            else:
                better = m4 < rm
                rm = jnp.minimum(m4, rm)
                rj = jnp.where(better, j4, rj)

        b = lax.rem(i, 2)
        rm_s[b] = rm
        rj_s[b] = rj


def _argmin_codes(flat_x2_bf, x_sq, emb_bf, y_sq):
    n = flat_x2_bf.shape[0]
    nblocks = n // _MBLK
    grid = (nblocks + 1,)
    codes, dsum = pl.pallas_call(
        _argmin_body,
        grid=grid,
        in_specs=[
            pl.BlockSpec((_MBLK, _DIM),
                         lambda i: (jnp.minimum(i, nblocks - 1), 0)),
            pl.BlockSpec((_MBLK, 1),
                         lambda i: (jnp.minimum(i, nblocks - 1), 0)),
            pl.BlockSpec((_NUM_EMB, _DIM), lambda i: (0, 0)),
            pl.BlockSpec((1, _NUM_EMB), lambda i: (0, 0)),
        ],
        out_specs=[
            pl.BlockSpec((_MBLK, 1), lambda i: (jnp.maximum(i, 1) - 1, 0)),
            pl.BlockSpec(memory_space=pltpu.SMEM),
        ],
        out_shape=[
            jax.ShapeDtypeStruct((n, 1), jnp.int32),
            jax.ShapeDtypeStruct((1, 1), jnp.float32),
        ],
        scratch_shapes=[
            pltpu.SMEM((1, 1), jnp.float32),
            pltpu.VMEM((2, _MBLK, _NBLK), jnp.float32),
            pltpu.VMEM((2, _MBLK, _NBLK), jnp.int32),
        ],
    )(flat_x2_bf, x_sq, emb_bf, y_sq)
    return codes.reshape(n), dsum[0, 0]


_SC_CHUNK = 96


def _make_gather(n_rows):
    info = plsc.get_sparse_core_info()
    nw = info.num_cores * info.num_subcores
    per_w = n_rows // nw
    n_chunks = per_w // _SC_CHUNK
    mesh = plsc.VectorSubcoreMesh(core_axis_name="c", subcore_axis_name="s")

    @functools.partial(
        pl.kernel,
        out_type=jax.ShapeDtypeStruct((n_rows, _DIM), jnp.float32),
        mesh=mesh,
        scratch_types=[
            pltpu.VMEM((per_w,), jnp.int32),
            pltpu.VMEM((_SC_CHUNK, _DIM), jnp.float32),
            pltpu.VMEM((_SC_CHUNK, _DIM), jnp.float32),
            pltpu.SemaphoreType.DMA,
            pltpu.SemaphoreType.DMA,
            pltpu.SemaphoreType.DMA,
            pltpu.SemaphoreType.DMA,
        ],
    )
    def gather(table_hbm, idx_hbm, out_hbm, idx_all, rows_a, rows_b,
               gsem_a, gsem_b, ssem_a, ssem_b):
        wid = lax.axis_index("s") * info.num_cores + lax.axis_index("c")
        base_w = wid * per_w
        pltpu.sync_copy(idx_hbm.at[pl.ds(base_w, per_w)], idx_all)
        bufs = (rows_a, rows_b)
        gsems = (gsem_a, gsem_b)
        ssems = (ssem_a, ssem_b)

        def start_gather(c):
            b = c % 2
            return pltpu.async_copy(
                table_hbm.at[idx_all.at[pl.ds(c * _SC_CHUNK, _SC_CHUNK)]],
                bufs[b], gsems[b])

        gs = [start_gather(0), start_gather(1)]
        stores = [None, None]
        for c in range(n_chunks):
            b = c % 2
            gs[b].wait()
            stores[b] = pltpu.async_copy(
                bufs[b],
                out_hbm.at[pl.ds(base_w + c * _SC_CHUNK, _SC_CHUNK)],
                ssems[b])
            if c + 2 < n_chunks:
                stores[b].wait()
                gs[b] = start_gather(c + 2)
        stores[(n_chunks - 2) % 2].wait()
        stores[(n_chunks - 1) % 2].wait()

    return gather


def kernel(inputs, embedding):
    orig_shape = inputs.shape
    flat_x = inputs.reshape(-1, _DIM)
    n = flat_x.shape[0]
    x_sq = jnp.sum(flat_x ** 2, axis=-1, keepdims=True)
    y_sq = jnp.sum(embedding ** 2, axis=-1)[None, :]
    flat_x2_bf = (flat_x + flat_x).astype(jnp.bfloat16)
    emb_bf = embedding.astype(jnp.bfloat16)

    half = n // 2
    gather_half = _make_gather(half)
    codes1, dsum1 = _argmin_codes(flat_x2_bf[:half], x_sq[:half],
                                  emb_bf, y_sq)
    q1 = gather_half(embedding, codes1)
    codes2, dsum2 = _argmin_codes(flat_x2_bf[half:], x_sq[half:],
                                  emb_bf, y_sq)
    q2 = gather_half(embedding, codes2)
    codes_flat = jnp.concatenate([codes1, codes2])
    quantized = jnp.concatenate([q1, q2])
    dist_sum = dsum1 + dsum2

    mse = dist_sum / (n * _DIM)
    commitment_loss = _COMMIT * mse
    codebook_loss = mse
    quantized_st = quantized.reshape(orig_shape)
    codes = codes_flat.reshape(orig_shape[:-1])
    return (quantized_st, codes, commitment_loss, codebook_loss)


# speedup vs baseline: 1.1135x; 1.1135x over previous
import functools

import jax
import jax.numpy as jnp
from jax import lax
from jax.experimental import pallas as pl
from jax.experimental.pallas import tpu as pltpu
from jax.experimental.pallas import tpu_sc as plsc

_NUM_EMB = 8192
_DIM = 256
_COMMIT = 0.25

_MBLK = 512
_NBLK = 512
_NSTEPS = _NUM_EMB // _NBLK


def _argmin_body(x_ref, xsq_ref, emb_ref, ysq_ref, codes_ref, dsum_ref,
                 acc_ref, rm_s, rj_s):
    i = pl.program_id(0)
    last = pl.num_programs(0) - 1

    @pl.when(i > 0)
    def _():
        b = lax.rem(i - 1, 2)
        rm = rm_s[b]
        rj = rj_s[b]
        run_min = jnp.min(rm, axis=1, keepdims=True)
        lanes = lax.broadcasted_iota(jnp.int32, rm.shape, 1)
        code_vec = rj * _NBLK + lanes
        run_idx = jnp.min(jnp.where(rm == run_min, code_vec, _NUM_EMB),
                          axis=1, keepdims=True)
        codes_ref[...] = run_idx
        prev = jnp.where(i == 1, 0.0, acc_ref[0, 0])
        acc_ref[0, 0] = prev + jnp.sum(run_min)

        @pl.when(i == last)
        def _():
            dsum_ref[0, 0] = acc_ref[0, 0]

    @pl.when(i < last)
    def _():
        x2 = x_ref[...]
        xsq = xsq_ref[...]

        def dist(j):
            e = emb_ref[pl.ds(j * _NBLK, _NBLK), :]
            ysq = ysq_ref[:, pl.ds(j * _NBLK, _NBLK)]
            s2 = lax.dot_general(x2, e, (((1,), (1,)), ((), ())),
                                 preferred_element_type=jnp.float32)
            return (xsq + ysq) - s2

        rm = None
        rj = None
        for k in range(_NSTEPS // 2):
            d0 = dist(2 * k)
            d1 = dist(2 * k + 1)
            m2 = jnp.minimum(d0, d1)
            j2 = jnp.where(d1 < d0, 2 * k + 1, 2 * k).astype(jnp.int32)
            if rm is None:
                rm, rj = m2, j2
            else:
                better = m2 < rm
                rm = jnp.minimum(m2, rm)
                rj = jnp.where(better, j2, rj)

        b = lax.rem(i, 2)
        rm_s[b] = rm
        rj_s[b] = rj


def _argmin_codes(flat_x2_bf, x_sq, emb_bf, y_sq):
    n = flat_x2_bf.shape[0]
    nblocks = n // _MBLK
    grid = (nblocks + 1,)
    codes, dsum = pl.pallas_call(
        _argmin_body,
        grid=grid,
        in_specs=[
            pl.BlockSpec((_MBLK, _DIM),
                         lambda i: (jnp.minimum(i, nblocks - 1), 0)),
            pl.BlockSpec((_MBLK, 1),
                         lambda i: (jnp.minimum(i, nblocks - 1), 0)),
            pl.BlockSpec((_NUM_EMB, _DIM), lambda i: (0, 0)),
            pl.BlockSpec((1, _NUM_EMB), lambda i: (0, 0)),
        ],
        out_specs=[
            pl.BlockSpec((_MBLK, 1), lambda i: (jnp.maximum(i, 1) - 1, 0)),
            pl.BlockSpec(memory_space=pltpu.SMEM),
        ],
        out_shape=[
            jax.ShapeDtypeStruct((n, 1), jnp.int32),
            jax.ShapeDtypeStruct((1, 1), jnp.float32),
        ],
        scratch_shapes=[
            pltpu.SMEM((1, 1), jnp.float32),
            pltpu.VMEM((2, _MBLK, _NBLK), jnp.float32),
            pltpu.VMEM((2, _MBLK, _NBLK), jnp.int32),
        ],
    )(flat_x2_bf, x_sq, emb_bf, y_sq)
    return codes.reshape(n), dsum[0, 0]


_SC_CHUNK = 96


def _make_gather(n_rows):
    info = plsc.get_sparse_core_info()
    nw = info.num_cores * info.num_subcores
    per_w = n_rows // nw
    n_chunks = per_w // _SC_CHUNK
    mesh = plsc.VectorSubcoreMesh(core_axis_name="c", subcore_axis_name="s")

    @functools.partial(
        pl.kernel,
        out_type=jax.ShapeDtypeStruct((n_rows, _DIM), jnp.float32),
        mesh=mesh,
        scratch_types=[
            pltpu.VMEM((per_w,), jnp.int32),
            pltpu.VMEM((_SC_CHUNK, _DIM), jnp.float32),
            pltpu.VMEM((_SC_CHUNK, _DIM), jnp.float32),
            pltpu.SemaphoreType.DMA,
            pltpu.SemaphoreType.DMA,
            pltpu.SemaphoreType.DMA,
            pltpu.SemaphoreType.DMA,
        ],
    )
    def gather(table_hbm, idx_hbm, out_hbm, idx_all, rows_a, rows_b,
               gsem_a, gsem_b, ssem_a, ssem_b):
        wid = lax.axis_index("s") * info.num_cores + lax.axis_index("c")
        base_w = wid * per_w
        pltpu.sync_copy(idx_hbm.at[pl.ds(base_w, per_w)], idx_all)
        bufs = (rows_a, rows_b)
        gsems = (gsem_a, gsem_b)
        ssems = (ssem_a, ssem_b)

        def start_gather(c):
            b = c % 2
            return pltpu.async_copy(
                table_hbm.at[idx_all.at[pl.ds(c * _SC_CHUNK, _SC_CHUNK)]],
                bufs[b], gsems[b])

        gs = [start_gather(0), start_gather(1)]
        stores = [None, None]
        for c in range(n_chunks):
            b = c % 2
            gs[b].wait()
            stores[b] = pltpu.async_copy(
                bufs[b],
                out_hbm.at[pl.ds(base_w + c * _SC_CHUNK, _SC_CHUNK)],
                ssems[b])
            if c + 2 < n_chunks:
                stores[b].wait()
                gs[b] = start_gather(c + 2)
        stores[(n_chunks - 2) % 2].wait()
        stores[(n_chunks - 1) % 2].wait()

    return gather


def kernel(inputs, embedding):
    orig_shape = inputs.shape
    flat_x = inputs.reshape(-1, _DIM)
    n = flat_x.shape[0]
    x_sq = jnp.sum(flat_x ** 2, axis=-1, keepdims=True)
    y_sq = jnp.sum(embedding ** 2, axis=-1)[None, :]
    flat_x2_bf = (flat_x + flat_x).astype(jnp.bfloat16)
    emb_bf = embedding.astype(jnp.bfloat16)

    codes_flat, dist_sum = _argmin_codes(flat_x2_bf, x_sq, emb_bf, y_sq)
    quantized = _make_gather(n)(embedding, codes_flat)

    mse = dist_sum / (n * _DIM)
    commitment_loss = _COMMIT * mse
    codebook_loss = mse
    quantized_st = quantized.reshape(orig_shape)
    codes = codes_flat.reshape(orig_shape[:-1])
    return (quantized_st, codes, commitment_loss, codebook_loss)


# speedup vs baseline: 1.2727x; 1.1430x over previous
import functools

import jax
import jax.numpy as jnp
from jax import lax
from jax.experimental import pallas as pl
from jax.experimental.pallas import tpu as pltpu
from jax.experimental.pallas import tpu_sc as plsc

_NUM_EMB = 8192
_DIM = 256
_COMMIT = 0.25

_MBLK = 2048
_NBLK = 256
_NSTEPS = _NUM_EMB // _NBLK


def _argmin_body(x_ref, xsq_ref, emb_ref, ysq_ref, codes_ref, dsum_ref,
                 acc_ref, rm_s, rj_s):
    i = pl.program_id(0)
    last = pl.num_programs(0) - 1

    @pl.when(i > 0)
    def _():
        b = lax.rem(i - 1, 2)
        rm = rm_s[b]
        rj = rj_s[b]
        run_min = jnp.min(rm, axis=1, keepdims=True)
        lanes = lax.broadcasted_iota(jnp.int32, rm.shape, 1)
        code_vec = rj * _NBLK + lanes
        run_idx = jnp.min(jnp.where(rm == run_min, code_vec, _NUM_EMB),
                          axis=1, keepdims=True)
        codes_ref[...] = run_idx
        prev = jnp.where(i == 1, 0.0, acc_ref[0, 0])
        acc_ref[0, 0] = prev + jnp.sum(run_min)

        @pl.when(i == last)
        def _():
            dsum_ref[0, 0] = acc_ref[0, 0]

    @pl.when(i < last)
    def _():
        x2 = x_ref[...]
        xsq = xsq_ref[...]

        def dist(j):
            e = emb_ref[pl.ds(j * _NBLK, _NBLK), :]
            ysq = ysq_ref[:, pl.ds(j * _NBLK, _NBLK)]
            s2 = lax.dot_general(x2, e, (((1,), (1,)), ((), ())),
                                 preferred_element_type=jnp.float32)
            return (xsq + ysq) - s2

        rm = None
        rj = None
        for k in range(_NSTEPS // 2):
            d0 = dist(2 * k)
            d1 = dist(2 * k + 1)
            m2 = jnp.minimum(d0, d1)
            j2 = jnp.where(d1 < d0, 2 * k + 1, 2 * k).astype(jnp.int32)
            if rm is None:
                rm, rj = m2, j2
            else:
                better = m2 < rm
                rm = jnp.minimum(m2, rm)
                rj = jnp.where(better, j2, rj)

        b = lax.rem(i, 2)
        rm_s[b] = rm
        rj_s[b] = rj


def _argmin_codes(flat_x2_bf, x_sq, emb_bf, y_sq):
    n = flat_x2_bf.shape[0]
    nblocks = n // _MBLK
    grid = (nblocks + 1,)
    codes, dsum = pl.pallas_call(
        _argmin_body,
        grid=grid,
        in_specs=[
            pl.BlockSpec((_MBLK, _DIM),
                         lambda i: (jnp.minimum(i, nblocks - 1), 0)),
            pl.BlockSpec((_MBLK, 1),
                         lambda i: (jnp.minimum(i, nblocks - 1), 0)),
            pl.BlockSpec((_NUM_EMB, _DIM), lambda i: (0, 0)),
            pl.BlockSpec((1, _NUM_EMB), lambda i: (0, 0)),
        ],
        out_specs=[
            pl.BlockSpec((_MBLK, 1), lambda i: (jnp.maximum(i, 1) - 1, 0)),
            pl.BlockSpec(memory_space=pltpu.SMEM),
        ],
        out_shape=[
            jax.ShapeDtypeStruct((n, 1), jnp.int32),
            jax.ShapeDtypeStruct((1, 1), jnp.float32),
        ],
        scratch_shapes=[
            pltpu.SMEM((1, 1), jnp.float32),
            pltpu.VMEM((2, _MBLK, _NBLK), jnp.float32),
            pltpu.VMEM((2, _MBLK, _NBLK), jnp.int32),
        ],
    )(flat_x2_bf, x_sq, emb_bf, y_sq)
    return codes.reshape(n), dsum[0, 0]


_SC_CHUNK = 96


def _make_gather(n_rows):
    info = plsc.get_sparse_core_info()
    nw = info.num_cores * info.num_subcores
    per_w = n_rows // nw
    n_chunks = per_w // _SC_CHUNK
    mesh = plsc.VectorSubcoreMesh(core_axis_name="c", subcore_axis_name="s")

    @functools.partial(
        pl.kernel,
        out_type=jax.ShapeDtypeStruct((n_rows, _DIM), jnp.float32),
        mesh=mesh,
        scratch_types=[
            pltpu.VMEM((per_w,), jnp.int32),
            pltpu.VMEM((_SC_CHUNK, _DIM), jnp.float32),
            pltpu.VMEM((_SC_CHUNK, _DIM), jnp.float32),
            pltpu.SemaphoreType.DMA,
            pltpu.SemaphoreType.DMA,
            pltpu.SemaphoreType.DMA,
            pltpu.SemaphoreType.DMA,
        ],
    )
    def gather(table_hbm, idx_hbm, out_hbm, idx_all, rows_a, rows_b,
               gsem_a, gsem_b, ssem_a, ssem_b):
        wid = lax.axis_index("s") * info.num_cores + lax.axis_index("c")
        base_w = wid * per_w
        pltpu.sync_copy(idx_hbm.at[pl.ds(base_w, per_w)], idx_all)
        bufs = (rows_a, rows_b)
        gsems = (gsem_a, gsem_b)
        ssems = (ssem_a, ssem_b)

        def start_gather(c):
            b = c % 2
            return pltpu.async_copy(
                table_hbm.at[idx_all.at[pl.ds(c * _SC_CHUNK, _SC_CHUNK)]],
                bufs[b], gsems[b])

        gs = [start_gather(0), start_gather(1)]
        stores = [None, None]
        for c in range(n_chunks):
            b = c % 2
            gs[b].wait()
            stores[b] = pltpu.async_copy(
                bufs[b],
                out_hbm.at[pl.ds(base_w + c * _SC_CHUNK, _SC_CHUNK)],
                ssems[b])
            if c + 2 < n_chunks:
                stores[b].wait()
                gs[b] = start_gather(c + 2)
        stores[(n_chunks - 2) % 2].wait()
        stores[(n_chunks - 1) % 2].wait()

    return gather


def kernel(inputs, embedding):
    orig_shape = inputs.shape
    flat_x = inputs.reshape(-1, _DIM)
    n = flat_x.shape[0]
    x_sq = jnp.sum(flat_x ** 2, axis=-1, keepdims=True)
    y_sq = jnp.sum(embedding ** 2, axis=-1)[None, :]
    flat_x2_bf = (flat_x + flat_x).astype(jnp.bfloat16)
    emb_bf = embedding.astype(jnp.bfloat16)

    codes_flat, dist_sum = _argmin_codes(flat_x2_bf, x_sq, emb_bf, y_sq)
    quantized = _make_gather(n)(embedding, codes_flat)

    mse = dist_sum / (n * _DIM)
    commitment_loss = _COMMIT * mse
    codebook_loss = mse
    quantized_st = quantized.reshape(orig_shape)
    codes = codes_flat.reshape(orig_shape[:-1])
    return (quantized_st, codes, commitment_loss, codebook_loss)
